# Initial kernel scaffold; baseline (speedup 1.0000x reference)
import functools

import jax
import jax.numpy as jnp
from jax import lax
from jax.experimental import pallas as pl
from jax.experimental.pallas import tpu as pltpu
from jax.experimental.pallas import tpu_sc as plsc

_MEAN = 3.5
_VOCAB = 1000
_BATCH = 16384
_TBL = 1008

_info = plsc.get_sparse_core_info()
_NC, _NS, _L = _info.num_cores, _info.num_subcores, _info.num_lanes
_NW = _NC * _NS
_CHUNK = _BATCH // _NW

_mesh = plsc.VectorSubcoreMesh(core_axis_name="c", subcore_axis_name="s")


@functools.partial(
    pl.kernel,
    mesh=_mesh,
    out_type=jax.ShapeDtypeStruct((_BATCH,), jnp.float32),
    scratch_types=[
        pltpu.VMEM((_CHUNK,), jnp.int32),
        pltpu.VMEM((_CHUNK,), jnp.int32),
        pltpu.VMEM((_TBL,), jnp.float32),
        pltpu.VMEM((_TBL,), jnp.float32),
        pltpu.VMEM((_CHUNK,), jnp.float32),
    ],
)
def _mean_model_sc(uid_hbm, mid_hbm, ut_hbm, mt_hbm, out_hbm,
                   uid_v, mid_v, ut_v, mt_v, out_v):
    wid = lax.axis_index("s") * _NC + lax.axis_index("c")
    base = wid * _CHUNK
    pltpu.sync_copy(ut_hbm, ut_v)
    pltpu.sync_copy(mt_hbm, mt_v)
    pltpu.sync_copy(uid_hbm.at[pl.ds(base, _CHUNK)], uid_v)
    pltpu.sync_copy(mid_hbm.at[pl.ds(base, _CHUNK)], mid_v)

    for i in range(_CHUNK // _L):
        sl = pl.ds(i * _L, _L)
        u = uid_v[sl]
        m = mid_v[sl]
        iu = jnp.where((u >= 0) & (u < _VOCAB), u + 1, 0)
        im = jnp.where((m >= 0) & (m < _VOCAB), m + 1, 0)
        eu = plsc.load_gather(ut_v, [iu])
        em = plsc.load_gather(mt_v, [im])
        out_v[sl] = eu + em + jnp.float32(_MEAN)

    pltpu.sync_copy(out_v, out_hbm.at[pl.ds(base, _CHUNK)])


def kernel(userId, movieId, user_table, movie_table):
    uid = userId.reshape(_BATCH)
    mid = movieId.reshape(_BATCH)
    ut = jnp.pad(user_table.reshape(-1), (0, _TBL - _VOCAB - 1))
    mt = jnp.pad(movie_table.reshape(-1), (0, _TBL - _VOCAB - 1))
    out = _mean_model_sc(uid, mid, ut, mt)
    return out.reshape(_BATCH, 1, 1)


# speedup vs baseline: 8.4256x; 8.4256x over previous
import functools

import jax
import jax.numpy as jnp
from jax import lax
from jax.experimental import pallas as pl
from jax.experimental.pallas import tpu as pltpu
from jax.experimental.pallas import tpu_sc as plsc

_MEAN = 3.5
_VOCAB = 1000
_BATCH = 16384
_TBL = 1008

_info = plsc.get_sparse_core_info()
_NC, _NS, _L = _info.num_cores, _info.num_subcores, _info.num_lanes
_NW = _NC * _NS
_CHUNK = _BATCH // _NW

_mesh = plsc.VectorSubcoreMesh(core_axis_name="c", subcore_axis_name="s")


@functools.partial(
    pl.kernel,
    mesh=_mesh,
    out_type=jax.ShapeDtypeStruct((_BATCH,), jnp.float32),
    compiler_params=pltpu.CompilerParams(needs_layout_passes=False),
    scratch_types=[
        pltpu.VMEM((_CHUNK,), jnp.int32),
        pltpu.VMEM((_CHUNK,), jnp.int32),
        pltpu.VMEM((_TBL,), jnp.float32),
        pltpu.VMEM((_TBL,), jnp.float32),
        pltpu.VMEM((_CHUNK,), jnp.float32),
    ],
)
def _mean_model_sc(uid_hbm, mid_hbm, ut_hbm, mt_hbm, out_hbm,
                   uid_v, mid_v, ut_v, mt_v, out_v):
    wid = lax.axis_index("s") * _NC + lax.axis_index("c")
    base = wid * _CHUNK
    pltpu.sync_copy(ut_hbm, ut_v)
    pltpu.sync_copy(mt_hbm, mt_v)
    pltpu.sync_copy(uid_hbm.at[pl.ds(base, _CHUNK)], uid_v)
    pltpu.sync_copy(mid_hbm.at[pl.ds(base, _CHUNK)], mid_v)

    for i in range(_CHUNK // _L):
        sl = pl.ds(i * _L, _L)
        u = uid_v[sl]
        m = mid_v[sl]
        iu = jnp.where((u >= 0) & (u < _VOCAB), u + 1, 0)
        im = jnp.where((m >= 0) & (m < _VOCAB), m + 1, 0)
        eu = plsc.load_gather(ut_v, [iu])
        em = plsc.load_gather(mt_v, [im])
        out_v[sl] = eu + em + jnp.float32(_MEAN)

    pltpu.sync_copy(out_v, out_hbm.at[pl.ds(base, _CHUNK)])


def kernel(userId, movieId, user_table, movie_table):
    uid = userId.reshape(_BATCH)
    mid = movieId.reshape(_BATCH)
    ut = jnp.pad(user_table.reshape(-1), (0, _TBL - _VOCAB - 1))
    mt = jnp.pad(movie_table.reshape(-1), (0, _TBL - _VOCAB - 1))
    out = _mean_model_sc(uid, mid, ut, mt)
    return out.reshape(_BATCH, 1, 1)


# speedup vs baseline: 9.1038x; 1.0805x over previous
import functools

import jax
import jax.numpy as jnp
from jax import lax
from jax.experimental import pallas as pl
from jax.experimental.pallas import tpu as pltpu
from jax.experimental.pallas import tpu_sc as plsc

_MEAN = 3.5
_VOCAB = 1000
_BATCH = 16384
_TBL = 1008

_info = plsc.get_sparse_core_info()
_NC, _NS, _L = _info.num_cores, _info.num_subcores, _info.num_lanes
_NW = _NC * _NS
_CHUNK = _BATCH // _NW

_mesh = plsc.VectorSubcoreMesh(core_axis_name="c", subcore_axis_name="s")


@functools.partial(
    pl.kernel,
    mesh=_mesh,
    out_type=jax.ShapeDtypeStruct((_BATCH,), jnp.float32),
    compiler_params=pltpu.CompilerParams(needs_layout_passes=False),
    scratch_types=[
        pltpu.VMEM((_CHUNK,), jnp.int32),
        pltpu.VMEM((_CHUNK,), jnp.int32),
        pltpu.VMEM((2 * _TBL,), jnp.float32),
        pltpu.VMEM((_CHUNK,), jnp.float32),
        pltpu.SemaphoreType.DMA,
    ],
)
def _mean_model_sc(uid_hbm, mid_hbm, tbl_hbm, out_hbm,
                   uid_v, mid_v, tbl_v, out_v, sem):
    wid = lax.axis_index("s") * _NC + lax.axis_index("c")
    base = wid * _CHUNK
    cp_t = pltpu.async_copy(tbl_hbm, tbl_v, sem)
    cp_u = pltpu.async_copy(uid_hbm.at[pl.ds(base, _CHUNK)], uid_v, sem)
    cp_m = pltpu.async_copy(mid_hbm.at[pl.ds(base, _CHUNK)], mid_v, sem)
    cp_t.wait()
    cp_u.wait()
    cp_m.wait()

    for i in range(_CHUNK // _L):
        sl = pl.ds(i * _L, _L)
        u = uid_v[sl]
        m = mid_v[sl]
        iu = jnp.where((u >= 0) & (u < _VOCAB), u + 1, 0)
        im = jnp.where((m >= 0) & (m < _VOCAB), m + 1, 0)
        eu = plsc.load_gather(tbl_v, [iu])
        em = plsc.load_gather(tbl_v, [im + _TBL])
        out_v[sl] = eu + em + jnp.float32(_MEAN)

    pltpu.sync_copy(out_v, out_hbm.at[pl.ds(base, _CHUNK)])


def kernel(userId, movieId, user_table, movie_table):
    uid = userId.reshape(_BATCH)
    mid = movieId.reshape(_BATCH)
    pad = jnp.zeros((_TBL - _VOCAB - 1,), jnp.float32)
    tbl = jnp.concatenate(
        [user_table.reshape(-1), pad, movie_table.reshape(-1), pad])
    out = _mean_model_sc(uid, mid, tbl)
    return out.reshape(_BATCH, 1, 1)


# speedup vs baseline: 9.1516x; 1.0052x over previous
import functools

import jax
import jax.numpy as jnp
from jax import lax
from jax.experimental import pallas as pl
from jax.experimental.pallas import tpu as pltpu
from jax.experimental.pallas import tpu_sc as plsc

_MEAN = 3.5
_VOCAB = 1000
_BATCH = 16384
_TBL = 1008

_info = plsc.get_sparse_core_info()
_NC, _NS, _L = _info.num_cores, _info.num_subcores, _info.num_lanes
_NW = _NC * _NS
_CHUNK = _BATCH // _NW

_mesh = plsc.VectorSubcoreMesh(core_axis_name="c", subcore_axis_name="s")


@functools.partial(
    pl.kernel,
    mesh=_mesh,
    out_type=jax.ShapeDtypeStruct((_BATCH,), jnp.float32),
    compiler_params=pltpu.CompilerParams(
        needs_layout_passes=False,
        disable_bounds_checks=True,
        disable_semaphore_checks=True,
        skip_device_barrier=True,
    ),
    scratch_types=[
        pltpu.VMEM((_CHUNK,), jnp.int32),
        pltpu.VMEM((_CHUNK,), jnp.int32),
        pltpu.VMEM((2 * _TBL,), jnp.float32),
        pltpu.VMEM((_CHUNK,), jnp.float32),
        pltpu.SemaphoreType.DMA,
    ],
)
def _mean_model_sc(uid_hbm, mid_hbm, tbl_hbm, out_hbm,
                   uid_v, mid_v, tbl_v, out_v, sem):
    wid = lax.axis_index("s") * _NC + lax.axis_index("c")
    base = wid * _CHUNK
    cp_t = pltpu.async_copy(tbl_hbm, tbl_v, sem)
    cp_u = pltpu.async_copy(uid_hbm.at[pl.ds(base, _CHUNK)], uid_v, sem)
    cp_m = pltpu.async_copy(mid_hbm.at[pl.ds(base, _CHUNK)], mid_v, sem)
    cp_t.wait()
    cp_u.wait()
    cp_m.wait()

    for i in range(_CHUNK // _L):
        sl = pl.ds(i * _L, _L)
        u = uid_v[sl]
        m = mid_v[sl]
        iu = jnp.where((u >= 0) & (u < _VOCAB), u + 1, 0)
        im = jnp.where((m >= 0) & (m < _VOCAB), m + 1, 0)
        eu = plsc.load_gather(tbl_v, [iu])
        em = plsc.load_gather(tbl_v, [im + _TBL])
        out_v[sl] = eu + em + jnp.float32(_MEAN)

    pltpu.sync_copy(out_v, out_hbm.at[pl.ds(base, _CHUNK)])


def kernel(userId, movieId, user_table, movie_table):
    uid = userId.reshape(_BATCH)
    mid = movieId.reshape(_BATCH)
    pad = jnp.zeros((_TBL - _VOCAB - 1,), jnp.float32)
    tbl = jnp.concatenate(
        [user_table.reshape(-1), pad, movie_table.reshape(-1), pad])
    out = _mean_model_sc(uid, mid, tbl)
    return out.reshape(_BATCH, 1, 1)
